# baseline (speedup 1.0000x reference)
import functools

import jax
import jax.numpy as jnp
from jax import lax
from jax.experimental import pallas as pl
from jax.experimental.pallas import tpu as pltpu
from jax.experimental.pallas import tpu_sc as plsc

_B, _J, _H, _W = 128, 17, 64, 48
_HW = _H * _W
_TOPK = 8

_NC, _NS, _L = 2, 16, 16
_NW = _NC * _NS
_RPW = _HW // _NW
_NLG = _B // _L

_sc_mesh = plsc.VectorSubcoreMesh(core_axis_name="c", subcore_axis_name="s")


@functools.partial(
    pl.kernel,
    out_type=jax.ShapeDtypeStruct((_NW, _J, _B), jnp.float32),
    mesh=_sc_mesh,
    scratch_types=[
        pltpu.VMEM((_RPW, _B), jnp.float32),
        pltpu.VMEM((_RPW, _B), jnp.float32),
        pltpu.VMEM((_J, _B), jnp.float32),
    ],
)
def _sc_partials(o_hbm, t_hbm, out_hbm, o_v, t_v, acc_v):
    wid = lax.axis_index("s") * _NC + lax.axis_index("c")
    base = wid * _RPW

    def per_joint(j, carry):
        pltpu.sync_copy(o_hbm.at[j, pl.ds(base, _RPW), :], o_v)
        pltpu.sync_copy(t_hbm.at[j, pl.ds(base, _RPW), :], t_v)

        def per_row(r, accs):
            new = []
            for k in range(_NLG):
                o16 = o_v[r, pl.ds(k * _L, _L)]
                t16 = t_v[r, pl.ds(k * _L, _L)]
                dd = o16 - t16
                new.append(accs[k] + dd * dd)
            return tuple(new)

        accs = lax.fori_loop(
            0, _RPW, per_row,
            tuple(jnp.zeros((_L,), jnp.float32) for _ in range(_NLG)),
        )
        for k in range(_NLG):
            acc_v[j, pl.ds(k * _L, _L)] = accs[k]
        return carry

    lax.fori_loop(0, _J, per_joint, 0)
    pltpu.sync_copy(acc_v, out_hbm.at[wid])


def _combine_kernel(p_ref, w_ref, out_ref):
    s = jnp.sum(p_ref[...], axis=0)
    w = w_ref[...]
    vals = s * (w * w) * (0.5 / _HW)
    row = jax.lax.broadcasted_iota(jnp.int32, vals.shape, 0)
    acc = jnp.zeros((_B,), jnp.float32)
    neg_inf = jnp.float32(-jnp.inf)
    for _ in range(_TOPK):
        m = jnp.max(vals, axis=0)
        acc = acc + m
        is_max = vals == m[None, :]
        first_idx = jnp.min(jnp.where(is_max, row, _J), axis=0)
        vals = jnp.where(row == first_idx[None, :], neg_inf, vals)
    out_ref[0, 0] = jnp.sum(acc) * (1.0 / (_TOPK * _B))


def kernel(outs, targets, target_weights):
    o = jnp.transpose(outs, (1, 2, 3, 0)).reshape(_J, _HW, _B)
    t = jnp.transpose(targets, (1, 2, 3, 0)).reshape(_J, _HW, _B)
    w = jnp.transpose(target_weights, (1, 2, 0)).reshape(_J, _B)
    part = _sc_partials(o, t)
    out = pl.pallas_call(
        _combine_kernel,
        out_specs=pl.BlockSpec(
            (1, 1), lambda: (0, 0), memory_space=pltpu.SMEM
        ),
        out_shape=jax.ShapeDtypeStruct((1, 1), jnp.float32),
    )(part, w)
    return out.reshape(())


# speedup vs baseline: 2.0843x; 2.0843x over previous
import functools

import jax
import jax.numpy as jnp
from jax import lax
from jax.experimental import pallas as pl
from jax.experimental.pallas import tpu as pltpu
from jax.experimental.pallas import tpu_sc as plsc

_B, _J, _H, _W = 128, 17, 64, 48
_HW = _H * _W
_TOPK = 8

_TCJ = 14
_SCJ = _J - _TCJ
_RB = 512
_GRID = _HW // _RB

_NC, _NS, _L = 2, 16, 16
_NW = _NC * _NS
_RPW = _HW // _NW
_NLG = _B // _L

_sc_mesh = plsc.VectorSubcoreMesh(core_axis_name="c", subcore_axis_name="s")


@functools.partial(
    pl.kernel,
    out_type=jax.ShapeDtypeStruct((_NW, _SCJ, _B), jnp.float32),
    mesh=_sc_mesh,
    scratch_types=[
        pltpu.VMEM((_RPW, _B), jnp.float32),
        pltpu.VMEM((_RPW, _B), jnp.float32),
        pltpu.VMEM((_SCJ, _B), jnp.float32),
    ],
)
def _sc_partials(o_hbm, t_hbm, out_hbm, o_v, t_v, acc_v):
    wid = lax.axis_index("s") * _NC + lax.axis_index("c")
    base = wid * _RPW

    def per_joint(j, carry):
        pltpu.sync_copy(o_hbm.at[_TCJ + j, pl.ds(base, _RPW), :], o_v)
        pltpu.sync_copy(t_hbm.at[_TCJ + j, pl.ds(base, _RPW), :], t_v)

        def per_row(r, accs):
            new = []
            for k in range(_NLG):
                o16 = o_v[r, pl.ds(k * _L, _L)]
                t16 = t_v[r, pl.ds(k * _L, _L)]
                dd = o16 - t16
                new.append(accs[k] + dd * dd)
            return tuple(new)

        accs = lax.fori_loop(
            0, _RPW, per_row,
            tuple(jnp.zeros((_L,), jnp.float32) for _ in range(_NLG)),
        )
        for k in range(_NLG):
            acc_v[j, pl.ds(k * _L, _L)] = accs[k]
        return carry

    lax.fori_loop(0, _SCJ, per_joint, 0)
    pltpu.sync_copy(acc_v, out_hbm.at[wid])


def _tc_sums_kernel(o_ref, t_ref, out_ref):
    i = pl.program_id(0)
    d = o_ref[...] - t_ref[...]
    part = jnp.sum(d * d, axis=1)

    @pl.when(i == 0)
    def _():
        out_ref[...] = jnp.zeros((_TCJ, _B), jnp.float32)

    out_ref[...] += part


def _combine_kernel(s_ref, p_ref, w_ref, out_ref):
    s_sc = jnp.sum(p_ref[...], axis=0)
    s = jnp.concatenate([s_ref[...], s_sc], axis=0)
    w = w_ref[...]
    vals = s * (w * w) * (0.5 / _HW)
    row = jax.lax.broadcasted_iota(jnp.int32, vals.shape, 0)
    acc = jnp.zeros((_B,), jnp.float32)
    neg_inf = jnp.float32(-jnp.inf)
    for _ in range(_TOPK):
        m = jnp.max(vals, axis=0)
        acc = acc + m
        is_max = vals == m[None, :]
        first_idx = jnp.min(jnp.where(is_max, row, _J), axis=0)
        vals = jnp.where(row == first_idx[None, :], neg_inf, vals)
    out_ref[0, 0] = jnp.sum(acc) * (1.0 / (_TOPK * _B))


def kernel(outs, targets, target_weights):
    o = jnp.transpose(outs, (1, 2, 3, 0)).reshape(_J, _HW, _B)
    t = jnp.transpose(targets, (1, 2, 3, 0)).reshape(_J, _HW, _B)
    w = jnp.transpose(target_weights, (1, 2, 0)).reshape(_J, _B)
    part = _sc_partials(o, t)
    s_tc = pl.pallas_call(
        _tc_sums_kernel,
        grid=(_GRID,),
        in_specs=[
            pl.BlockSpec((_TCJ, _RB, _B), lambda i: (0, i, 0)),
            pl.BlockSpec((_TCJ, _RB, _B), lambda i: (0, i, 0)),
        ],
        out_specs=pl.BlockSpec((_TCJ, _B), lambda i: (0, 0)),
        out_shape=jax.ShapeDtypeStruct((_TCJ, _B), jnp.float32),
    )(o, t)
    out = pl.pallas_call(
        _combine_kernel,
        out_specs=pl.BlockSpec(
            (1, 1), lambda: (0, 0), memory_space=pltpu.SMEM
        ),
        out_shape=jax.ShapeDtypeStruct((1, 1), jnp.float32),
    )(s_tc, part, w)
    return out.reshape(())


# speedup vs baseline: 3.5689x; 1.7123x over previous
import jax
import jax.numpy as jnp
from jax.experimental import pallas as pl
from jax.experimental.pallas import tpu as pltpu

_B, _J, _H, _W = 128, 17, 64, 48
_HW = _H * _W
_RB = 256
_GRID = _HW // _RB
_TOPK = 8


def _ohkm_kernel(o_ref, t_ref, w_ref, out_ref, s_ref):
    i = pl.program_id(0)
    d = o_ref[...] - t_ref[...]
    part = jnp.sum(d * d, axis=1)

    @pl.when(i == 0)
    def _():
        s_ref[...] = jnp.zeros((_J, _B), jnp.float32)

    s_ref[...] += part

    @pl.when(i == _GRID - 1)
    def _():
        w = w_ref[...]
        vals = s_ref[...] * (w * w) * (0.5 / _HW)
        row = jax.lax.broadcasted_iota(jnp.int32, vals.shape, 0)
        acc = jnp.zeros((_B,), jnp.float32)
        neg_inf = jnp.float32(-jnp.inf)
        for _ in range(_TOPK):
            m = jnp.max(vals, axis=0)
            acc = acc + m
            is_max = vals == m[None, :]
            first_idx = jnp.min(jnp.where(is_max, row, _J), axis=0)
            vals = jnp.where(row == first_idx[None, :], neg_inf, vals)
        out_ref[0, 0] = jnp.sum(acc) * (1.0 / (_TOPK * _B))


def kernel(outs, targets, target_weights):
    o = jnp.transpose(outs, (1, 2, 3, 0)).reshape(_J, _HW, _B)
    t = jnp.transpose(targets, (1, 2, 3, 0)).reshape(_J, _HW, _B)
    w = jnp.transpose(target_weights, (1, 2, 0)).reshape(_J, _B)
    out = pl.pallas_call(
        _ohkm_kernel,
        grid=(_GRID,),
        in_specs=[
            pl.BlockSpec((_J, _RB, _B), lambda i: (0, i, 0)),
            pl.BlockSpec((_J, _RB, _B), lambda i: (0, i, 0)),
            pl.BlockSpec((_J, _B), lambda i: (0, 0)),
        ],
        out_specs=pl.BlockSpec(
            (1, 1), lambda i: (0, 0), memory_space=pltpu.SMEM
        ),
        out_shape=jax.ShapeDtypeStruct((1, 1), jnp.float32),
        scratch_shapes=[pltpu.VMEM((_J, _B), jnp.float32)],
    )(o, t, w)
    return out.reshape(())


# speedup vs baseline: 3.7080x; 1.0390x over previous
import jax
import jax.numpy as jnp
from jax.experimental import pallas as pl
from jax.experimental.pallas import tpu as pltpu

_B, _J, _H, _W = 128, 17, 64, 48
_HW = _H * _W
_ROWS = _J * _HW
_GRID = 6
_RB = _ROWS // _GRID
_SEG = _RB // _J
_SPJ = _HW // _SEG
_NSEG = _GRID * _J
_TOPK = 8


def _ohkm_kernel(o_ref, t_ref, w_ref, out_ref, s_ref):
    i = pl.program_id(0)
    d = o_ref[...] - t_ref[...]
    part = jnp.sum((d * d).reshape(_J, _SEG, _B), axis=1)
    s_ref[pl.ds(i, 1)] = part[None]

    @pl.when(i == _GRID - 1)
    def _():
        rows = []
        for j in range(_J):
            g0 = j * _SPJ
            r = s_ref[g0 // _J, g0 % _J, :]
            for m in range(1, _SPJ):
                g = g0 + m
                r = r + s_ref[g // _J, g % _J, :]
            rows.append(r)
        s = jnp.stack(rows, axis=0)
        w = w_ref[...]
        vals = s * (w * w) * (0.5 / _HW)
        row = jax.lax.broadcasted_iota(jnp.int32, vals.shape, 0)
        acc = jnp.zeros((_B,), jnp.float32)
        neg_inf = jnp.float32(-jnp.inf)
        for _ in range(_TOPK):
            m = jnp.max(vals, axis=0)
            acc = acc + m
            is_max = vals == m[None, :]
            first_idx = jnp.min(jnp.where(is_max, row, _J), axis=0)
            vals = jnp.where(row == first_idx[None, :], neg_inf, vals)
        out_ref[0, 0] = jnp.sum(acc) * (1.0 / (_TOPK * _B))


def kernel(outs, targets, target_weights):
    o = jnp.transpose(outs, (1, 2, 3, 0)).reshape(_ROWS, _B)
    t = jnp.transpose(targets, (1, 2, 3, 0)).reshape(_ROWS, _B)
    w = jnp.transpose(target_weights, (1, 2, 0)).reshape(_J, _B)
    out = pl.pallas_call(
        _ohkm_kernel,
        grid=(_GRID,),
        in_specs=[
            pl.BlockSpec((_RB, _B), lambda i: (i, 0)),
            pl.BlockSpec((_RB, _B), lambda i: (i, 0)),
            pl.BlockSpec((_J, _B), lambda i: (0, 0)),
        ],
        out_specs=pl.BlockSpec(
            (1, 1), lambda i: (0, 0), memory_space=pltpu.SMEM
        ),
        out_shape=jax.ShapeDtypeStruct((1, 1), jnp.float32),
        scratch_shapes=[pltpu.VMEM((_GRID, _J, _B), jnp.float32)],
    )(o, t, w)
    return out.reshape(())
